# baseline (speedup 1.0000x reference)
import functools

import jax
import jax.numpy as jnp
from jax.experimental import pallas as pl
from jax.experimental.pallas import tpu as pltpu

_K = 100
_FOCAL = 5000.0
_RING = 8


def _topk_kernel(logits_ref, scores_ref, qidx_ref, labels_ref, *, num_classes):
    x = logits_ref[...]
    p = jax.nn.sigmoid(x)
    b, nc = p.shape
    col = jax.lax.broadcasted_iota(jnp.int32, (b, nc), 1)
    kcol = jax.lax.broadcasted_iota(jnp.int32, (b, _K), 1)

    def body(i, carry):
        p, sc, qi, lb = carry
        m = jnp.max(p, axis=1, keepdims=True)
        idx = jnp.min(jnp.where(p == m, col, nc), axis=1, keepdims=True)
        sel = kcol == i
        sc = jnp.where(sel, m, sc)
        qi = jnp.where(sel, idx // num_classes, qi)
        lb = jnp.where(sel, idx % num_classes, lb)
        p = jnp.where(col == idx, -1.0, p)
        return p, sc, qi, lb

    sc0 = jnp.zeros((b, _K), jnp.float32)
    qi0 = jnp.zeros((b, _K), jnp.int32)
    lb0 = jnp.zeros((b, _K), jnp.int32)
    _, sc, qi, lb = jax.lax.fori_loop(0, _K, body, (p, sc0, qi0, lb0))
    scores_ref[...] = sc
    qidx_ref[...] = qi
    labels_ref[...] = lb


def _gather_kernel(idx_ref, meta_ref,
                   boxes_ref, pose_ref, beta_ref, expr_ref, cam_ref,
                   kx_ref, ky_ref, kz_ref, verts_hbm,
                   boxes_out, pose_out, beta_out, expr_out, cam_out,
                   kxg_out, kyg_out, kzg_out, px_out, py_out, verts_out,
                   sems):
    b = pl.program_id(0)

    def gk(k, _):
        q = idx_ref[b, k]
        pose_out[0, pl.ds(k, 1), :] = pose_ref[0, pl.ds(q, 1), :]
        beta_out[0, pl.ds(k, 1), :] = beta_ref[0, pl.ds(q, 1), :]
        expr_out[0, pl.ds(k, 1), :] = expr_ref[0, pl.ds(q, 1), :]
        cam_out[0, pl.ds(k, 1), :] = cam_ref[0, pl.ds(q, 1), :]
        boxes_out[0, pl.ds(k, 1), :] = boxes_ref[0, pl.ds(q, 1), :]
        kxg_out[0, pl.ds(k, 1), :] = kx_ref[0, pl.ds(q, 1), :]
        kyg_out[0, pl.ds(k, 1), :] = ky_ref[0, pl.ds(q, 1), :]
        kzg_out[0, pl.ds(k, 1), :] = kz_ref[0, pl.ds(q, 1), :]

        @pl.when(k >= _RING)
        def _wait_prev():
            pltpu.make_async_copy(verts_hbm.at[b, 0], verts_out.at[b, 0],
                                  sems.at[k % _RING]).wait()

        pltpu.make_async_copy(verts_hbm.at[b, q], verts_out.at[b, k],
                              sems.at[k % _RING]).start()
        return 0

    jax.lax.fori_loop(0, _K, gk, 0)
    for j in range(_RING):
        pltpu.make_async_copy(verts_hbm.at[b, 0], verts_out.at[b, 0],
                              sems.at[(_K - _RING + j) % _RING]).wait()

    th = meta_ref[b, 0]
    tw = meta_ref[b, 1]
    v = boxes_out[...]
    cx = v[..., 0:1]
    cy = v[..., 1:2]
    w = v[..., 2:3]
    h = v[..., 3:4]
    boxes_out[...] = jnp.concatenate(
        [(cx - 0.5 * w) * tw, (cy - 0.5 * h) * th,
         (cx + 0.5 * w) * tw, (cy + 0.5 * h) * th], axis=-1)

    ih = meta_ref[b, 2]
    iw = meta_ref[b, 3]
    camg = cam_out[...]
    s = camg[..., 0:1]
    tx = camg[..., 1:2]
    ty = camg[..., 2:3]
    tz = 2.0 * _FOCAL / (iw * s + 1e-9)
    zq = kzg_out[...] + tz + 1e-9
    px_out[...] = (kxg_out[...] + tx) / zq * _FOCAL + iw * 0.5
    py_out[...] = (kyg_out[...] + ty) / zq * _FOCAL + ih * 0.5


def kernel(pred_logits, pred_boxes, pred_smpl_fullpose, pred_smpl_beta,
           pred_smpl_expr, pred_smpl_cam, pred_smpl_kp3d, pred_smpl_verts,
           target_sizes, img_shape):
    B, N, C = pred_logits.shape
    KP = pred_smpl_kp3d.shape[2]
    V = pred_smpl_verts.shape[2]
    P = pred_smpl_fullpose.shape[2]

    scores, qidx, labels = pl.pallas_call(
        functools.partial(_topk_kernel, num_classes=C),
        out_shape=[
            jax.ShapeDtypeStruct((B, _K), jnp.float32),
            jax.ShapeDtypeStruct((B, _K), jnp.int32),
            jax.ShapeDtypeStruct((B, _K), jnp.int32),
        ],
    )(pred_logits.reshape(B, N * C))

    meta = jnp.concatenate([target_sizes, img_shape], axis=1)
    kx = pred_smpl_kp3d[..., 0]
    ky = pred_smpl_kp3d[..., 1]
    kz = pred_smpl_kp3d[..., 2]

    def brow(d):
        return pl.BlockSpec((1, N, d), lambda bb, idx, mt: (bb, 0, 0))

    def orow(d):
        return pl.BlockSpec((1, _K, d), lambda bb, idx, mt: (bb, 0, 0))

    grid_spec = pltpu.PrefetchScalarGridSpec(
        num_scalar_prefetch=2,
        grid=(B,),
        in_specs=[
            brow(4),
            brow(P),
            brow(10),
            brow(10),
            brow(3),
            brow(KP),
            brow(KP),
            brow(KP),
            pl.BlockSpec(memory_space=pltpu.MemorySpace.HBM),
        ],
        out_specs=[
            orow(4),
            orow(P),
            orow(10),
            orow(10),
            orow(3),
            orow(KP),
            orow(KP),
            orow(KP),
            orow(KP),
            orow(KP),
            pl.BlockSpec(memory_space=pltpu.MemorySpace.HBM),
        ],
        scratch_shapes=[pltpu.SemaphoreType.DMA((_RING,))],
    )
    outs = pl.pallas_call(
        _gather_kernel,
        grid_spec=grid_spec,
        out_shape=[
            jax.ShapeDtypeStruct((B, _K, 4), jnp.float32),
            jax.ShapeDtypeStruct((B, _K, P), jnp.float32),
            jax.ShapeDtypeStruct((B, _K, 10), jnp.float32),
            jax.ShapeDtypeStruct((B, _K, 10), jnp.float32),
            jax.ShapeDtypeStruct((B, _K, 3), jnp.float32),
            jax.ShapeDtypeStruct((B, _K, KP), jnp.float32),
            jax.ShapeDtypeStruct((B, _K, KP), jnp.float32),
            jax.ShapeDtypeStruct((B, _K, KP), jnp.float32),
            jax.ShapeDtypeStruct((B, _K, KP), jnp.float32),
            jax.ShapeDtypeStruct((B, _K, KP), jnp.float32),
            jax.ShapeDtypeStruct((B, _K, V, 3), jnp.float32),
        ],
    )(qidx, meta, pred_boxes, pred_smpl_fullpose, pred_smpl_beta,
      pred_smpl_expr, pred_smpl_cam, kx, ky, kz, pred_smpl_verts)

    (boxes, pose_o, beta_o, expr_o, cam_o,
     kxg, kyg, kzg, px, py, verts_o) = outs
    kp2d = jnp.stack([px, py], axis=-1)
    kp3d_o = jnp.stack([kxg, kyg, kzg], axis=-1)
    return (scores, labels, boxes, kp2d, pose_o, beta_o, expr_o, cam_o,
            kp3d_o, verts_o)


# speedup vs baseline: 32.7699x; 32.7699x over previous
import functools

import jax
import jax.numpy as jnp
from jax.experimental import pallas as pl
from jax.experimental.pallas import tpu as pltpu

_K = 100
_FOCAL = 5000.0
_RING = 8


def _topk_kernel(logits_ref, scores_ref, qidx_ref, labels_ref, *, num_classes):
    x = logits_ref[...]
    p = jax.nn.sigmoid(x)
    b, nc = p.shape
    col = jax.lax.broadcasted_iota(jnp.int32, (b, nc), 1)
    kcol = jax.lax.broadcasted_iota(jnp.int32, (b, _K), 1)

    def body(i, carry):
        p, sc, qi, lb = carry
        m = jnp.max(p, axis=1, keepdims=True)
        idx = jnp.min(jnp.where(p == m, col, nc), axis=1, keepdims=True)
        sel = kcol == i
        sc = jnp.where(sel, m, sc)
        qi = jnp.where(sel, idx // num_classes, qi)
        lb = jnp.where(sel, idx % num_classes, lb)
        p = jnp.where(col == idx, -1.0, p)
        return p, sc, qi, lb

    sc0 = jnp.zeros((b, _K), jnp.float32)
    qi0 = jnp.zeros((b, _K), jnp.int32)
    lb0 = jnp.zeros((b, _K), jnp.int32)
    _, sc, qi, lb = jax.lax.fori_loop(0, _K, body, (p, sc0, qi0, lb0))
    scores_ref[...] = sc
    qidx_ref[...] = qi
    labels_ref[...] = lb


def _gather_kernel(idx_ref, meta_ref,
                   boxes_ref, pose_ref, beta_ref, expr_ref, cam_ref,
                   kx_ref, ky_ref, kz_ref, vx_hbm, vy_hbm, vz_hbm,
                   boxes_out, pose_out, beta_out, expr_out, cam_out,
                   kxg_out, kyg_out, kzg_out, px_out, py_out,
                   vxo_hbm, vyo_hbm, vzo_hbm,
                   sems):
    vsrc = (vx_hbm, vy_hbm, vz_hbm)
    vdst = (vxo_hbm, vyo_hbm, vzo_hbm)
    b = pl.program_id(0)

    def gk(k, _):
        q = idx_ref[b, k]
        pose_out[0, pl.ds(k, 1), :] = pose_ref[0, pl.ds(q, 1), :]
        beta_out[0, pl.ds(k, 1), :] = beta_ref[0, pl.ds(q, 1), :]
        expr_out[0, pl.ds(k, 1), :] = expr_ref[0, pl.ds(q, 1), :]
        cam_out[0, pl.ds(k, 1), :] = cam_ref[0, pl.ds(q, 1), :]
        boxes_out[0, pl.ds(k, 1), :] = boxes_ref[0, pl.ds(q, 1), :]
        kxg_out[0, pl.ds(k, 1), :] = kx_ref[0, pl.ds(q, 1), :]
        kyg_out[0, pl.ds(k, 1), :] = ky_ref[0, pl.ds(q, 1), :]
        kzg_out[0, pl.ds(k, 1), :] = kz_ref[0, pl.ds(q, 1), :]

        for c in range(3):
            m = 3 * k + c

            @pl.when(m >= _RING)
            def _wait_prev(c=c):
                pltpu.make_async_copy(
                    vsrc[c].at[pl.ds(b, 1), pl.ds(0, 1), :],
                    vdst[c].at[pl.ds(b, 1), pl.ds(0, 1), :],
                    sems.at[m % _RING]).wait()

            pltpu.make_async_copy(
                vsrc[c].at[pl.ds(b, 1), pl.ds(q, 1), :],
                vdst[c].at[pl.ds(b, 1), pl.ds(k, 1), :],
                sems.at[m % _RING]).start()
        return 0

    jax.lax.fori_loop(0, _K, gk, 0)
    for j in range(_RING):
        pltpu.make_async_copy(
            vx_hbm.at[pl.ds(0, 1), pl.ds(0, 1), :],
            vxo_hbm.at[pl.ds(0, 1), pl.ds(0, 1), :],
            sems.at[(3 * _K - _RING + j) % _RING]).wait()

    th = meta_ref[b, 0]
    tw = meta_ref[b, 1]
    v = boxes_out[...]
    cx = v[..., 0:1]
    cy = v[..., 1:2]
    w = v[..., 2:3]
    h = v[..., 3:4]
    boxes_out[...] = jnp.concatenate(
        [(cx - 0.5 * w) * tw, (cy - 0.5 * h) * th,
         (cx + 0.5 * w) * tw, (cy + 0.5 * h) * th], axis=-1)

    ih = meta_ref[b, 2]
    iw = meta_ref[b, 3]
    camg = cam_out[...]
    s = camg[..., 0:1]
    tx = camg[..., 1:2]
    ty = camg[..., 2:3]
    tz = 2.0 * _FOCAL / (iw * s + 1e-9)
    zq = kzg_out[...] + tz + 1e-9
    px_out[...] = (kxg_out[...] + tx) / zq * _FOCAL + iw * 0.5
    py_out[...] = (kyg_out[...] + ty) / zq * _FOCAL + ih * 0.5


def kernel(pred_logits, pred_boxes, pred_smpl_fullpose, pred_smpl_beta,
           pred_smpl_expr, pred_smpl_cam, pred_smpl_kp3d, pred_smpl_verts,
           target_sizes, img_shape):
    B, N, C = pred_logits.shape
    KP = pred_smpl_kp3d.shape[2]
    V = pred_smpl_verts.shape[2]
    P = pred_smpl_fullpose.shape[2]

    scores, qidx, labels = pl.pallas_call(
        functools.partial(_topk_kernel, num_classes=C),
        out_shape=[
            jax.ShapeDtypeStruct((B, _K), jnp.float32),
            jax.ShapeDtypeStruct((B, _K), jnp.int32),
            jax.ShapeDtypeStruct((B, _K), jnp.int32),
        ],
    )(pred_logits.reshape(B, N * C))

    meta = jnp.concatenate([target_sizes, img_shape], axis=1)
    kx = pred_smpl_kp3d[..., 0]
    ky = pred_smpl_kp3d[..., 1]
    kz = pred_smpl_kp3d[..., 2]
    vx = pred_smpl_verts[..., 0]
    vy = pred_smpl_verts[..., 1]
    vz = pred_smpl_verts[..., 2]

    def brow(d):
        return pl.BlockSpec((1, N, d), lambda bb, idx, mt: (bb, 0, 0))

    def orow(d):
        return pl.BlockSpec((1, _K, d), lambda bb, idx, mt: (bb, 0, 0))

    grid_spec = pltpu.PrefetchScalarGridSpec(
        num_scalar_prefetch=2,
        grid=(B,),
        in_specs=[
            brow(4),
            brow(P),
            brow(10),
            brow(10),
            brow(3),
            brow(KP),
            brow(KP),
            brow(KP),
            pl.BlockSpec(memory_space=pltpu.MemorySpace.HBM),
            pl.BlockSpec(memory_space=pltpu.MemorySpace.HBM),
            pl.BlockSpec(memory_space=pltpu.MemorySpace.HBM),
        ],
        out_specs=[
            orow(4),
            orow(P),
            orow(10),
            orow(10),
            orow(3),
            orow(KP),
            orow(KP),
            orow(KP),
            orow(KP),
            orow(KP),
            pl.BlockSpec(memory_space=pltpu.MemorySpace.HBM),
            pl.BlockSpec(memory_space=pltpu.MemorySpace.HBM),
            pl.BlockSpec(memory_space=pltpu.MemorySpace.HBM),
        ],
        scratch_shapes=[pltpu.SemaphoreType.DMA((_RING,))],
    )
    outs = pl.pallas_call(
        _gather_kernel,
        grid_spec=grid_spec,
        out_shape=[
            jax.ShapeDtypeStruct((B, _K, 4), jnp.float32),
            jax.ShapeDtypeStruct((B, _K, P), jnp.float32),
            jax.ShapeDtypeStruct((B, _K, 10), jnp.float32),
            jax.ShapeDtypeStruct((B, _K, 10), jnp.float32),
            jax.ShapeDtypeStruct((B, _K, 3), jnp.float32),
            jax.ShapeDtypeStruct((B, _K, KP), jnp.float32),
            jax.ShapeDtypeStruct((B, _K, KP), jnp.float32),
            jax.ShapeDtypeStruct((B, _K, KP), jnp.float32),
            jax.ShapeDtypeStruct((B, _K, KP), jnp.float32),
            jax.ShapeDtypeStruct((B, _K, KP), jnp.float32),
            jax.ShapeDtypeStruct((B, _K, V), jnp.float32),
            jax.ShapeDtypeStruct((B, _K, V), jnp.float32),
            jax.ShapeDtypeStruct((B, _K, V), jnp.float32),
        ],
    )(qidx, meta, pred_boxes, pred_smpl_fullpose, pred_smpl_beta,
      pred_smpl_expr, pred_smpl_cam, kx, ky, kz, vx, vy, vz)

    (boxes, pose_o, beta_o, expr_o, cam_o,
     kxg, kyg, kzg, px, py, vxo, vyo, vzo) = outs
    kp2d = jnp.stack([px, py], axis=-1)
    kp3d_o = jnp.stack([kxg, kyg, kzg], axis=-1)
    verts_o = jnp.stack([vxo, vyo, vzo], axis=-1)
    return (scores, labels, boxes, kp2d, pose_o, beta_o, expr_o, cam_o,
            kp3d_o, verts_o)


# speedup vs baseline: 73.0119x; 2.2280x over previous
import functools

import jax
import jax.numpy as jnp
from jax.experimental import pallas as pl
from jax.experimental.pallas import tpu as pltpu

_K = 100
_FOCAL = 5000.0
_RING = 8


def _topk_kernel(logits_ref, scores_ref, qidx_ref, labels_ref, *, num_classes):
    x = logits_ref[...]
    p = jax.nn.sigmoid(x)
    b, nc = p.shape
    col = jax.lax.broadcasted_iota(jnp.int32, (b, nc), 1)
    kcol = jax.lax.broadcasted_iota(jnp.int32, (b, _K), 1)

    def body(i, carry):
        p, sc, qi, lb = carry
        m = jnp.max(p, axis=1, keepdims=True)
        idx = jnp.min(jnp.where(p == m, col, nc), axis=1, keepdims=True)
        sel = kcol == i
        sc = jnp.where(sel, m, sc)
        qi = jnp.where(sel, idx // num_classes, qi)
        lb = jnp.where(sel, idx % num_classes, lb)
        p = jnp.where(col == idx, -1.0, p)
        return p, sc, qi, lb

    sc0 = jnp.zeros((b, _K), jnp.float32)
    qi0 = jnp.zeros((b, _K), jnp.int32)
    lb0 = jnp.zeros((b, _K), jnp.int32)
    _, sc, qi, lb = jax.lax.fori_loop(0, _K, body, (p, sc0, qi0, lb0))
    scores_ref[...] = sc
    qidx_ref[...] = qi
    labels_ref[...] = lb


def _gather_kernel(idx_ref, meta_ref,
                   boxes_ref, pose_ref, beta_ref, expr_ref, cam_ref,
                   kx_ref, ky_ref, kz_ref,
                   boxes_out, pose_out, beta_out, expr_out, cam_out,
                   kxg_out, kyg_out, kzg_out, px_out, py_out):
    b = pl.program_id(0)

    def gk(k, _):
        q = idx_ref[b, k]
        pose_out[0, pl.ds(k, 1), :] = pose_ref[0, pl.ds(q, 1), :]
        beta_out[0, pl.ds(k, 1), :] = beta_ref[0, pl.ds(q, 1), :]
        expr_out[0, pl.ds(k, 1), :] = expr_ref[0, pl.ds(q, 1), :]
        cam_out[0, pl.ds(k, 1), :] = cam_ref[0, pl.ds(q, 1), :]
        boxes_out[0, pl.ds(k, 1), :] = boxes_ref[0, pl.ds(q, 1), :]
        kxg_out[0, pl.ds(k, 1), :] = kx_ref[0, pl.ds(q, 1), :]
        kyg_out[0, pl.ds(k, 1), :] = ky_ref[0, pl.ds(q, 1), :]
        kzg_out[0, pl.ds(k, 1), :] = kz_ref[0, pl.ds(q, 1), :]

        return 0

    jax.lax.fori_loop(0, _K, gk, 0)

    th = meta_ref[b, 0]
    tw = meta_ref[b, 1]
    v = boxes_out[...]
    cx = v[..., 0:1]
    cy = v[..., 1:2]
    w = v[..., 2:3]
    h = v[..., 3:4]
    boxes_out[...] = jnp.concatenate(
        [(cx - 0.5 * w) * tw, (cy - 0.5 * h) * th,
         (cx + 0.5 * w) * tw, (cy + 0.5 * h) * th], axis=-1)

    ih = meta_ref[b, 2]
    iw = meta_ref[b, 3]
    camg = cam_out[...]
    s = camg[..., 0:1]
    tx = camg[..., 1:2]
    ty = camg[..., 2:3]
    tz = 2.0 * _FOCAL / (iw * s + 1e-9)
    zq = kzg_out[...] + tz + 1e-9
    px_out[...] = (kxg_out[...] + tx) / zq * _FOCAL + iw * 0.5
    py_out[...] = (kyg_out[...] + ty) / zq * _FOCAL + ih * 0.5


def _verts_kernel(idx_ref, vx_ref, vy_ref, vz_ref,
                  vxo_ref, vyo_ref, vzo_ref):
    b = pl.program_id(0)
    k = pl.program_id(1)
    r = idx_ref[b, k] % 8
    d = k % 8
    vxo_ref[0, 0, pl.ds(d, 1), :] = vx_ref[0, pl.ds(r, 1), :]
    vyo_ref[0, 0, pl.ds(d, 1), :] = vy_ref[0, pl.ds(r, 1), :]
    vzo_ref[0, 0, pl.ds(d, 1), :] = vz_ref[0, pl.ds(r, 1), :]


def kernel(pred_logits, pred_boxes, pred_smpl_fullpose, pred_smpl_beta,
           pred_smpl_expr, pred_smpl_cam, pred_smpl_kp3d, pred_smpl_verts,
           target_sizes, img_shape):
    B, N, C = pred_logits.shape
    KP = pred_smpl_kp3d.shape[2]
    V = pred_smpl_verts.shape[2]
    P = pred_smpl_fullpose.shape[2]

    scores, qidx, labels = pl.pallas_call(
        functools.partial(_topk_kernel, num_classes=C),
        out_shape=[
            jax.ShapeDtypeStruct((B, _K), jnp.float32),
            jax.ShapeDtypeStruct((B, _K), jnp.int32),
            jax.ShapeDtypeStruct((B, _K), jnp.int32),
        ],
    )(pred_logits.reshape(B, N * C))

    meta = jnp.concatenate([target_sizes, img_shape], axis=1)
    kx = pred_smpl_kp3d[..., 0]
    ky = pred_smpl_kp3d[..., 1]
    kz = pred_smpl_kp3d[..., 2]
    vx = pred_smpl_verts[..., 0]
    vy = pred_smpl_verts[..., 1]
    vz = pred_smpl_verts[..., 2]

    def brow(d):
        return pl.BlockSpec((1, N, d), lambda bb, idx, mt: (bb, 0, 0))

    def orow(d):
        return pl.BlockSpec((1, _K, d), lambda bb, idx, mt: (bb, 0, 0))

    grid_spec = pltpu.PrefetchScalarGridSpec(
        num_scalar_prefetch=2,
        grid=(B,),
        in_specs=[
            brow(4),
            brow(P),
            brow(10),
            brow(10),
            brow(3),
            brow(KP),
            brow(KP),
            brow(KP),
        ],
        out_specs=[
            orow(4),
            orow(P),
            orow(10),
            orow(10),
            orow(3),
            orow(KP),
            orow(KP),
            orow(KP),
            orow(KP),
            orow(KP),
        ],
    )
    outs = pl.pallas_call(
        _gather_kernel,
        grid_spec=grid_spec,
        out_shape=[
            jax.ShapeDtypeStruct((B, _K, 4), jnp.float32),
            jax.ShapeDtypeStruct((B, _K, P), jnp.float32),
            jax.ShapeDtypeStruct((B, _K, 10), jnp.float32),
            jax.ShapeDtypeStruct((B, _K, 10), jnp.float32),
            jax.ShapeDtypeStruct((B, _K, 3), jnp.float32),
            jax.ShapeDtypeStruct((B, _K, KP), jnp.float32),
            jax.ShapeDtypeStruct((B, _K, KP), jnp.float32),
            jax.ShapeDtypeStruct((B, _K, KP), jnp.float32),
            jax.ShapeDtypeStruct((B, _K, KP), jnp.float32),
            jax.ShapeDtypeStruct((B, _K, KP), jnp.float32),
        ],
    )(qidx, meta, pred_boxes, pred_smpl_fullpose, pred_smpl_beta,
      pred_smpl_expr, pred_smpl_cam, kx, ky, kz)

    (boxes, pose_o, beta_o, expr_o, cam_o,
     kxg, kyg, kzg, px, py) = outs

    KG = (_K + 7) // 8
    vspec = pl.BlockSpec((1, 8, V), lambda bb, kk, idx: (bb, idx[bb, kk] // 8, 0))
    ospec = pl.BlockSpec((1, 1, 8, V), lambda bb, kk, idx: (bb, kk // 8, 0, 0))
    vgrid = pltpu.PrefetchScalarGridSpec(
        num_scalar_prefetch=1,
        grid=(B, _K),
        in_specs=[vspec, vspec, vspec],
        out_specs=[ospec, ospec, ospec],
    )
    vshape = jax.ShapeDtypeStruct((B, KG, 8, V), jnp.float32)
    vxo, vyo, vzo = pl.pallas_call(
        _verts_kernel,
        grid_spec=vgrid,
        out_shape=[vshape, vshape, vshape],
    )(qidx, vx, vy, vz)
    vxo = vxo.reshape(B, KG * 8, V)[:, :_K]
    vyo = vyo.reshape(B, KG * 8, V)[:, :_K]
    vzo = vzo.reshape(B, KG * 8, V)[:, :_K]
    kp2d = jnp.stack([px, py], axis=-1)
    kp3d_o = jnp.stack([kxg, kyg, kzg], axis=-1)
    verts_o = jnp.stack([vxo, vyo, vzo], axis=-1)
    return (scores, labels, boxes, kp2d, pose_o, beta_o, expr_o, cam_o,
            kp3d_o, verts_o)


# speedup vs baseline: 73.0425x; 1.0004x over previous
import functools

import jax
import jax.numpy as jnp
from jax.experimental import pallas as pl
from jax.experimental.pallas import tpu as pltpu

_K = 100
_FOCAL = 5000.0
_RING = 8


def _topk_kernel(logits_ref, scores_ref, qidx_ref, labels_ref, *, num_classes):
    x = logits_ref[...]
    p = jax.nn.sigmoid(x)
    b, nc = p.shape
    col = jax.lax.broadcasted_iota(jnp.int32, (b, nc), 1)
    kcol = jax.lax.broadcasted_iota(jnp.int32, (b, _K), 1)

    def body(i, carry):
        p, sc, qi, lb = carry
        m = jnp.max(p, axis=1, keepdims=True)
        idx = jnp.min(jnp.where(p == m, col, nc), axis=1, keepdims=True)
        sel = kcol == i
        sc = jnp.where(sel, m, sc)
        qi = jnp.where(sel, idx // num_classes, qi)
        lb = jnp.where(sel, idx % num_classes, lb)
        p = jnp.where(col == idx, -1.0, p)
        return p, sc, qi, lb

    sc0 = jnp.zeros((b, _K), jnp.float32)
    qi0 = jnp.zeros((b, _K), jnp.int32)
    lb0 = jnp.zeros((b, _K), jnp.int32)
    _, sc, qi, lb = jax.lax.fori_loop(0, _K, body, (p, sc0, qi0, lb0))
    scores_ref[...] = sc
    qidx_ref[...] = qi
    labels_ref[...] = lb


def _gather_kernel(idx_ref, meta_ref,
                   boxes_ref, pose_ref, beta_ref, expr_ref, cam_ref,
                   kx_ref, ky_ref, kz_ref,
                   boxes_out, pose_out, beta_out, expr_out, cam_out,
                   kxg_out, kyg_out, kzg_out, px_out, py_out):
    b = pl.program_id(0)

    def gk(k, _):
        q = idx_ref[b, k]
        pose_out[0, pl.ds(k, 1), :] = pose_ref[0, pl.ds(q, 1), :]
        beta_out[0, pl.ds(k, 1), :] = beta_ref[0, pl.ds(q, 1), :]
        expr_out[0, pl.ds(k, 1), :] = expr_ref[0, pl.ds(q, 1), :]
        cam_out[0, pl.ds(k, 1), :] = cam_ref[0, pl.ds(q, 1), :]
        boxes_out[0, pl.ds(k, 1), :] = boxes_ref[0, pl.ds(q, 1), :]
        kxg_out[0, pl.ds(k, 1), :] = kx_ref[0, pl.ds(q, 1), :]
        kyg_out[0, pl.ds(k, 1), :] = ky_ref[0, pl.ds(q, 1), :]
        kzg_out[0, pl.ds(k, 1), :] = kz_ref[0, pl.ds(q, 1), :]

        return 0

    jax.lax.fori_loop(0, _K, gk, 0)

    th = meta_ref[b, 0]
    tw = meta_ref[b, 1]
    v = boxes_out[...]
    cx = v[..., 0:1]
    cy = v[..., 1:2]
    w = v[..., 2:3]
    h = v[..., 3:4]
    boxes_out[...] = jnp.concatenate(
        [(cx - 0.5 * w) * tw, (cy - 0.5 * h) * th,
         (cx + 0.5 * w) * tw, (cy + 0.5 * h) * th], axis=-1)

    ih = meta_ref[b, 2]
    iw = meta_ref[b, 3]
    camg = cam_out[...]
    s = camg[..., 0:1]
    tx = camg[..., 1:2]
    ty = camg[..., 2:3]
    tz = 2.0 * _FOCAL / (iw * s + 1e-9)
    zq = kzg_out[...] + tz + 1e-9
    px_out[...] = (kxg_out[...] + tx) / zq * _FOCAL + iw * 0.5
    py_out[...] = (kyg_out[...] + ty) / zq * _FOCAL + ih * 0.5


def _verts_kernel(idx_ref, vx_ref, vy_ref, vz_ref,
                  vxo_ref, vyo_ref, vzo_ref):
    b = pl.program_id(0)
    k = pl.program_id(1)
    r = idx_ref[b, k] % 8
    d = k % 8
    vxo_ref[0, 0, pl.ds(d, 1), :] = vx_ref[0, pl.ds(r, 1), :]
    vyo_ref[0, 0, pl.ds(d, 1), :] = vy_ref[0, pl.ds(r, 1), :]
    vzo_ref[0, 0, pl.ds(d, 1), :] = vz_ref[0, pl.ds(r, 1), :]


def kernel(pred_logits, pred_boxes, pred_smpl_fullpose, pred_smpl_beta,
           pred_smpl_expr, pred_smpl_cam, pred_smpl_kp3d, pred_smpl_verts,
           target_sizes, img_shape):
    B, N, C = pred_logits.shape
    KP = pred_smpl_kp3d.shape[2]
    V = pred_smpl_verts.shape[2]
    P = pred_smpl_fullpose.shape[2]

    scores, qidx, labels = pl.pallas_call(
        functools.partial(_topk_kernel, num_classes=C),
        out_shape=[
            jax.ShapeDtypeStruct((B, _K), jnp.float32),
            jax.ShapeDtypeStruct((B, _K), jnp.int32),
            jax.ShapeDtypeStruct((B, _K), jnp.int32),
        ],
    )(pred_logits.reshape(B, N * C))

    meta = jnp.concatenate([target_sizes, img_shape], axis=1)
    kx = pred_smpl_kp3d[..., 0]
    ky = pred_smpl_kp3d[..., 1]
    kz = pred_smpl_kp3d[..., 2]
    vx = pred_smpl_verts[..., 0]
    vy = pred_smpl_verts[..., 1]
    vz = pred_smpl_verts[..., 2]

    def brow(d):
        return pl.BlockSpec((1, N, d), lambda bb, idx, mt: (bb, 0, 0))

    def orow(d):
        return pl.BlockSpec((1, _K, d), lambda bb, idx, mt: (bb, 0, 0))

    grid_spec = pltpu.PrefetchScalarGridSpec(
        num_scalar_prefetch=2,
        grid=(B,),
        in_specs=[
            brow(4),
            brow(P),
            brow(10),
            brow(10),
            brow(3),
            brow(KP),
            brow(KP),
            brow(KP),
        ],
        out_specs=[
            orow(4),
            orow(P),
            orow(10),
            orow(10),
            orow(3),
            orow(KP),
            orow(KP),
            orow(KP),
            orow(KP),
            orow(KP),
        ],
    )
    outs = pl.pallas_call(
        _gather_kernel,
        grid_spec=grid_spec,
        out_shape=[
            jax.ShapeDtypeStruct((B, _K, 4), jnp.float32),
            jax.ShapeDtypeStruct((B, _K, P), jnp.float32),
            jax.ShapeDtypeStruct((B, _K, 10), jnp.float32),
            jax.ShapeDtypeStruct((B, _K, 10), jnp.float32),
            jax.ShapeDtypeStruct((B, _K, 3), jnp.float32),
            jax.ShapeDtypeStruct((B, _K, KP), jnp.float32),
            jax.ShapeDtypeStruct((B, _K, KP), jnp.float32),
            jax.ShapeDtypeStruct((B, _K, KP), jnp.float32),
            jax.ShapeDtypeStruct((B, _K, KP), jnp.float32),
            jax.ShapeDtypeStruct((B, _K, KP), jnp.float32),
        ],
    )(qidx, meta, pred_boxes, pred_smpl_fullpose, pred_smpl_beta,
      pred_smpl_expr, pred_smpl_cam, kx, ky, kz)

    (boxes, pose_o, beta_o, expr_o, cam_o,
     kxg, kyg, kzg, px, py) = outs

    KG = (_K + 7) // 8
    vspec = pl.BlockSpec((1, 8, V), lambda bb, kk, idx: (bb, idx[bb, kk] // 8, 0))
    ospec = pl.BlockSpec((1, 1, 8, V), lambda bb, kk, idx: (bb, kk // 8, 0, 0))
    vgrid = pltpu.PrefetchScalarGridSpec(
        num_scalar_prefetch=1,
        grid=(B, _K),
        in_specs=[vspec, vspec, vspec],
        out_specs=[ospec, ospec, ospec],
    )
    vshape = jax.ShapeDtypeStruct((B, KG, 8, V), jnp.float32)
    vxo, vyo, vzo = pl.pallas_call(
        _verts_kernel,
        grid_spec=vgrid,
        out_shape=[vshape, vshape, vshape],
        compiler_params=pltpu.CompilerParams(
            dimension_semantics=("parallel", "arbitrary")),
    )(qidx, vx, vy, vz)
    vxo = vxo.reshape(B, KG * 8, V)[:, :_K]
    vyo = vyo.reshape(B, KG * 8, V)[:, :_K]
    vzo = vzo.reshape(B, KG * 8, V)[:, :_K]
    kp2d = jnp.stack([px, py], axis=-1)
    kp3d_o = jnp.stack([kxg, kyg, kzg], axis=-1)
    verts_o = jnp.stack([vxo, vyo, vzo], axis=-1)
    return (scores, labels, boxes, kp2d, pose_o, beta_o, expr_o, cam_o,
            kp3d_o, verts_o)


# speedup vs baseline: 86.2215x; 1.1804x over previous
import functools

import jax
import jax.numpy as jnp
from jax.experimental import pallas as pl
from jax.experimental.pallas import tpu as pltpu

_K = 100
_FOCAL = 5000.0
_RING = 8


def _topk_kernel(logits_ref, scores_ref, qidx_ref, labels_ref, *, num_classes):
    x = logits_ref[...]
    p = jax.nn.sigmoid(x)
    b, nc = p.shape
    col = jax.lax.broadcasted_iota(jnp.int32, (b, nc), 1)
    kcol = jax.lax.broadcasted_iota(jnp.int32, (b, _K), 1)

    def body(i, carry):
        p, sc, qi, lb = carry
        m = jnp.max(p, axis=1, keepdims=True)
        idx = jnp.min(jnp.where(p == m, col, nc), axis=1, keepdims=True)
        sel = kcol == i
        sc = jnp.where(sel, m, sc)
        qi = jnp.where(sel, idx // num_classes, qi)
        lb = jnp.where(sel, idx % num_classes, lb)
        p = jnp.where(col == idx, -1.0, p)
        return p, sc, qi, lb

    sc0 = jnp.zeros((b, _K), jnp.float32)
    qi0 = jnp.zeros((b, _K), jnp.int32)
    lb0 = jnp.zeros((b, _K), jnp.int32)
    _, sc, qi, lb = jax.lax.fori_loop(0, _K, body, (p, sc0, qi0, lb0))
    scores_ref[...] = sc
    qidx_ref[...] = qi
    labels_ref[...] = lb


def _gather_kernel(idx_ref, meta_ref,
                   boxes_ref, pose_ref, beta_ref, expr_ref, cam_ref,
                   kx_ref, ky_ref, kz_ref,
                   boxes_out, pose_out, beta_out, expr_out, cam_out,
                   kxg_out, kyg_out, kzg_out, px_out, py_out):
    b = pl.program_id(0)

    def gk(k, _):
        q = idx_ref[b, k]
        pose_out[0, pl.ds(k, 1), :] = pose_ref[0, pl.ds(q, 1), :]
        beta_out[0, pl.ds(k, 1), :] = beta_ref[0, pl.ds(q, 1), :]
        expr_out[0, pl.ds(k, 1), :] = expr_ref[0, pl.ds(q, 1), :]
        cam_out[0, pl.ds(k, 1), :] = cam_ref[0, pl.ds(q, 1), :]
        boxes_out[0, pl.ds(k, 1), :] = boxes_ref[0, pl.ds(q, 1), :]
        kxg_out[0, pl.ds(k, 1), :] = kx_ref[0, pl.ds(q, 1), :]
        kyg_out[0, pl.ds(k, 1), :] = ky_ref[0, pl.ds(q, 1), :]
        kzg_out[0, pl.ds(k, 1), :] = kz_ref[0, pl.ds(q, 1), :]

        return 0

    jax.lax.fori_loop(0, _K, gk, 0)

    th = meta_ref[b, 0]
    tw = meta_ref[b, 1]
    v = boxes_out[...]
    cx = v[..., 0:1]
    cy = v[..., 1:2]
    w = v[..., 2:3]
    h = v[..., 3:4]
    boxes_out[...] = jnp.concatenate(
        [(cx - 0.5 * w) * tw, (cy - 0.5 * h) * th,
         (cx + 0.5 * w) * tw, (cy + 0.5 * h) * th], axis=-1)

    ih = meta_ref[b, 2]
    iw = meta_ref[b, 3]
    camg = cam_out[...]
    s = camg[..., 0:1]
    tx = camg[..., 1:2]
    ty = camg[..., 2:3]
    tz = 2.0 * _FOCAL / (iw * s + 1e-9)
    zq = kzg_out[...] + tz + 1e-9
    px_out[...] = (kxg_out[...] + tx) / zq * _FOCAL + iw * 0.5
    py_out[...] = (kyg_out[...] + ty) / zq * _FOCAL + ih * 0.5


def _verts_kernel(idx_ref, *refs):
    b = pl.program_id(0)
    g = pl.program_id(1)
    ins = refs[:24]
    outs = refs[24:]
    for j in range(8):
        q = idx_ref[b, jnp.minimum(8 * g + j, _K - 1)]
        r = q % 8
        for c in range(3):
            outs[c][0, 0, pl.ds(j, 1), :] = ins[8 * c + j][0, pl.ds(r, 1), :]


def kernel(pred_logits, pred_boxes, pred_smpl_fullpose, pred_smpl_beta,
           pred_smpl_expr, pred_smpl_cam, pred_smpl_kp3d, pred_smpl_verts,
           target_sizes, img_shape):
    B, N, C = pred_logits.shape
    KP = pred_smpl_kp3d.shape[2]
    V = pred_smpl_verts.shape[2]
    P = pred_smpl_fullpose.shape[2]

    scores, qidx, labels = pl.pallas_call(
        functools.partial(_topk_kernel, num_classes=C),
        out_shape=[
            jax.ShapeDtypeStruct((B, _K), jnp.float32),
            jax.ShapeDtypeStruct((B, _K), jnp.int32),
            jax.ShapeDtypeStruct((B, _K), jnp.int32),
        ],
    )(pred_logits.reshape(B, N * C))

    meta = jnp.concatenate([target_sizes, img_shape], axis=1)
    kx = pred_smpl_kp3d[..., 0]
    ky = pred_smpl_kp3d[..., 1]
    kz = pred_smpl_kp3d[..., 2]
    vx = pred_smpl_verts[..., 0]
    vy = pred_smpl_verts[..., 1]
    vz = pred_smpl_verts[..., 2]

    def brow(d):
        return pl.BlockSpec((1, N, d), lambda bb, idx, mt: (bb, 0, 0))

    def orow(d):
        return pl.BlockSpec((1, _K, d), lambda bb, idx, mt: (bb, 0, 0))

    grid_spec = pltpu.PrefetchScalarGridSpec(
        num_scalar_prefetch=2,
        grid=(B,),
        in_specs=[
            brow(4),
            brow(P),
            brow(10),
            brow(10),
            brow(3),
            brow(KP),
            brow(KP),
            brow(KP),
        ],
        out_specs=[
            orow(4),
            orow(P),
            orow(10),
            orow(10),
            orow(3),
            orow(KP),
            orow(KP),
            orow(KP),
            orow(KP),
            orow(KP),
        ],
    )
    outs = pl.pallas_call(
        _gather_kernel,
        grid_spec=grid_spec,
        out_shape=[
            jax.ShapeDtypeStruct((B, _K, 4), jnp.float32),
            jax.ShapeDtypeStruct((B, _K, P), jnp.float32),
            jax.ShapeDtypeStruct((B, _K, 10), jnp.float32),
            jax.ShapeDtypeStruct((B, _K, 10), jnp.float32),
            jax.ShapeDtypeStruct((B, _K, 3), jnp.float32),
            jax.ShapeDtypeStruct((B, _K, KP), jnp.float32),
            jax.ShapeDtypeStruct((B, _K, KP), jnp.float32),
            jax.ShapeDtypeStruct((B, _K, KP), jnp.float32),
            jax.ShapeDtypeStruct((B, _K, KP), jnp.float32),
            jax.ShapeDtypeStruct((B, _K, KP), jnp.float32),
        ],
    )(qidx, meta, pred_boxes, pred_smpl_fullpose, pred_smpl_beta,
      pred_smpl_expr, pred_smpl_cam, kx, ky, kz)

    (boxes, pose_o, beta_o, expr_o, cam_o,
     kxg, kyg, kzg, px, py) = outs

    KG = (_K + 7) // 8

    def vspec(j):
        return pl.BlockSpec(
            (1, 8, V),
            lambda bb, gg, idx, j=j: (
                bb, idx[bb, jnp.minimum(8 * gg + j, _K - 1)] // 8, 0))

    ospec = pl.BlockSpec((1, 1, 8, V), lambda bb, gg, idx: (bb, gg, 0, 0))
    vgrid = pltpu.PrefetchScalarGridSpec(
        num_scalar_prefetch=1,
        grid=(B, KG),
        in_specs=[vspec(j) for j in range(8)] * 3,
        out_specs=[ospec, ospec, ospec],
    )
    vshape = jax.ShapeDtypeStruct((B, KG, 8, V), jnp.float32)
    vxo, vyo, vzo = pl.pallas_call(
        _verts_kernel,
        grid_spec=vgrid,
        out_shape=[vshape, vshape, vshape],
        compiler_params=pltpu.CompilerParams(
            dimension_semantics=("parallel", "arbitrary")),
    )(qidx, *([vx] * 8), *([vy] * 8), *([vz] * 8))
    vxo = vxo.reshape(B, KG * 8, V)[:, :_K]
    vyo = vyo.reshape(B, KG * 8, V)[:, :_K]
    vzo = vzo.reshape(B, KG * 8, V)[:, :_K]
    kp2d = jnp.stack([px, py], axis=-1)
    kp3d_o = jnp.stack([kxg, kyg, kzg], axis=-1)
    verts_o = jnp.stack([vxo, vyo, vzo], axis=-1)
    return (scores, labels, boxes, kp2d, pose_o, beta_o, expr_o, cam_o,
            kp3d_o, verts_o)


# speedup vs baseline: 86.3855x; 1.0019x over previous
import functools

import jax
import jax.numpy as jnp
from jax.experimental import pallas as pl
from jax.experimental.pallas import tpu as pltpu

_K = 100
_FOCAL = 5000.0


def _topk_kernel(logits_ref, scores_ref, qidx_ref, labels_ref, *, num_classes):
    x = logits_ref[...]
    p = jax.nn.sigmoid(x)
    b, nc = p.shape
    col = jax.lax.broadcasted_iota(jnp.int32, (b, nc), 1)
    kcol = jax.lax.broadcasted_iota(jnp.int32, (b, _K), 1)

    def body(i, carry):
        p, sc, qi, lb = carry
        m = jnp.max(p, axis=1, keepdims=True)
        idx = jnp.min(jnp.where(p == m, col, nc), axis=1, keepdims=True)
        sel = kcol == i
        sc = jnp.where(sel, m, sc)
        qi = jnp.where(sel, idx // num_classes, qi)
        lb = jnp.where(sel, idx % num_classes, lb)
        p = jnp.where(col == idx, -1.0, p)
        return p, sc, qi, lb

    sc0 = jnp.zeros((b, _K), jnp.float32)
    qi0 = jnp.zeros((b, _K), jnp.int32)
    lb0 = jnp.zeros((b, _K), jnp.int32)
    _, sc, qi, lb = jax.lax.fori_loop(0, _K, body, (p, sc0, qi0, lb0))
    scores_ref[...] = sc
    qidx_ref[...] = qi
    labels_ref[...] = lb


def _gather_kernel(idx_ref, meta_ref,
                   boxes_ref, pose_ref, beta_ref, expr_ref, cam_ref,
                   kx_ref, ky_ref, kz_ref,
                   boxes_out, pose_out, beta_out, expr_out, cam_out,
                   kxg_out, kyg_out, kzg_out, px_out, py_out):
    b = pl.program_id(0)

    def gk(k, _):
        q = idx_ref[b, k]
        pose_out[0, pl.ds(k, 1), :] = pose_ref[0, pl.ds(q, 1), :]
        beta_out[0, pl.ds(k, 1), :] = beta_ref[0, pl.ds(q, 1), :]
        expr_out[0, pl.ds(k, 1), :] = expr_ref[0, pl.ds(q, 1), :]
        cam_out[0, pl.ds(k, 1), :] = cam_ref[0, pl.ds(q, 1), :]
        boxes_out[0, pl.ds(k, 1), :] = boxes_ref[0, pl.ds(q, 1), :]
        kxg_out[0, pl.ds(k, 1), :] = kx_ref[0, pl.ds(q, 1), :]
        kyg_out[0, pl.ds(k, 1), :] = ky_ref[0, pl.ds(q, 1), :]
        kzg_out[0, pl.ds(k, 1), :] = kz_ref[0, pl.ds(q, 1), :]

        return 0

    jax.lax.fori_loop(0, _K, gk, 0)

    th = meta_ref[b, 0]
    tw = meta_ref[b, 1]
    v = boxes_out[...]
    cx = v[..., 0:1]
    cy = v[..., 1:2]
    w = v[..., 2:3]
    h = v[..., 3:4]
    boxes_out[...] = jnp.concatenate(
        [(cx - 0.5 * w) * tw, (cy - 0.5 * h) * th,
         (cx + 0.5 * w) * tw, (cy + 0.5 * h) * th], axis=-1)

    ih = meta_ref[b, 2]
    iw = meta_ref[b, 3]
    camg = cam_out[...]
    s = camg[..., 0:1]
    tx = camg[..., 1:2]
    ty = camg[..., 2:3]
    tz = 2.0 * _FOCAL / (iw * s + 1e-9)
    zq = kzg_out[...] + tz + 1e-9
    px_out[...] = (kxg_out[...] + tx) / zq * _FOCAL + iw * 0.5
    py_out[...] = (kyg_out[...] + ty) / zq * _FOCAL + ih * 0.5


def _verts_kernel(idx_ref, *refs):
    b = pl.program_id(0)
    g = pl.program_id(1)
    ins = refs[:24]
    outs = refs[24:]
    for j in range(8):
        q = idx_ref[b, jnp.minimum(8 * g + j, _K - 1)]
        r = q % 8
        for c in range(3):
            outs[c][0, 0, pl.ds(j, 1), :] = ins[8 * c + j][0, pl.ds(r, 1), :]


def kernel(pred_logits, pred_boxes, pred_smpl_fullpose, pred_smpl_beta,
           pred_smpl_expr, pred_smpl_cam, pred_smpl_kp3d, pred_smpl_verts,
           target_sizes, img_shape):
    B, N, C = pred_logits.shape
    KP = pred_smpl_kp3d.shape[2]
    V = pred_smpl_verts.shape[2]
    P = pred_smpl_fullpose.shape[2]

    scores, qidx, labels = pl.pallas_call(
        functools.partial(_topk_kernel, num_classes=C),
        out_shape=[
            jax.ShapeDtypeStruct((B, _K), jnp.float32),
            jax.ShapeDtypeStruct((B, _K), jnp.int32),
            jax.ShapeDtypeStruct((B, _K), jnp.int32),
        ],
    )(pred_logits.reshape(B, N * C))

    meta = jnp.concatenate([target_sizes, img_shape], axis=1)
    kx = pred_smpl_kp3d[..., 0]
    ky = pred_smpl_kp3d[..., 1]
    kz = pred_smpl_kp3d[..., 2]
    vx = pred_smpl_verts[..., 0]
    vy = pred_smpl_verts[..., 1]
    vz = pred_smpl_verts[..., 2]

    def brow(d):
        return pl.BlockSpec((1, N, d), lambda bb, idx, mt: (bb, 0, 0))

    def orow(d):
        return pl.BlockSpec((1, _K, d), lambda bb, idx, mt: (bb, 0, 0))

    grid_spec = pltpu.PrefetchScalarGridSpec(
        num_scalar_prefetch=2,
        grid=(B,),
        in_specs=[
            brow(4),
            brow(P),
            brow(10),
            brow(10),
            brow(3),
            brow(KP),
            brow(KP),
            brow(KP),
        ],
        out_specs=[
            orow(4),
            orow(P),
            orow(10),
            orow(10),
            orow(3),
            orow(KP),
            orow(KP),
            orow(KP),
            orow(KP),
            orow(KP),
        ],
    )
    outs = pl.pallas_call(
        _gather_kernel,
        grid_spec=grid_spec,
        out_shape=[
            jax.ShapeDtypeStruct((B, _K, 4), jnp.float32),
            jax.ShapeDtypeStruct((B, _K, P), jnp.float32),
            jax.ShapeDtypeStruct((B, _K, 10), jnp.float32),
            jax.ShapeDtypeStruct((B, _K, 10), jnp.float32),
            jax.ShapeDtypeStruct((B, _K, 3), jnp.float32),
            jax.ShapeDtypeStruct((B, _K, KP), jnp.float32),
            jax.ShapeDtypeStruct((B, _K, KP), jnp.float32),
            jax.ShapeDtypeStruct((B, _K, KP), jnp.float32),
            jax.ShapeDtypeStruct((B, _K, KP), jnp.float32),
            jax.ShapeDtypeStruct((B, _K, KP), jnp.float32),
        ],
    )(qidx, meta, pred_boxes, pred_smpl_fullpose, pred_smpl_beta,
      pred_smpl_expr, pred_smpl_cam, kx, ky, kz)

    (boxes, pose_o, beta_o, expr_o, cam_o,
     kxg, kyg, kzg, px, py) = outs

    KG = (_K + 7) // 8

    def vspec(j):
        return pl.BlockSpec(
            (1, 8, V),
            lambda bb, gg, idx, j=j: (
                bb, idx[bb, jnp.minimum(8 * gg + j, _K - 1)] // 8, 0))

    ospec = pl.BlockSpec((1, 1, 8, V), lambda bb, gg, idx: (bb, gg, 0, 0))
    vgrid = pltpu.PrefetchScalarGridSpec(
        num_scalar_prefetch=1,
        grid=(B, KG),
        in_specs=[vspec(j) for j in range(8)] * 3,
        out_specs=[ospec, ospec, ospec],
    )
    vshape = jax.ShapeDtypeStruct((B, KG, 8, V), jnp.float32)
    vxo, vyo, vzo = pl.pallas_call(
        _verts_kernel,
        grid_spec=vgrid,
        out_shape=[vshape, vshape, vshape],
        compiler_params=pltpu.CompilerParams(
            dimension_semantics=("parallel", "arbitrary")),
    )(qidx, *([vx] * 8), *([vy] * 8), *([vz] * 8))
    vxo = vxo.reshape(B, KG * 8, V)[:, :_K]
    vyo = vyo.reshape(B, KG * 8, V)[:, :_K]
    vzo = vzo.reshape(B, KG * 8, V)[:, :_K]
    kp2d = jnp.stack([px, py], axis=-1)
    kp3d_o = jnp.stack([kxg, kyg, kzg], axis=-1)
    verts_o = jnp.stack([vxo, vyo, vzo], axis=-1)
    return (scores, labels, boxes, kp2d, pose_o, beta_o, expr_o, cam_o,
            kp3d_o, verts_o)
